# Initial kernel scaffold; baseline (speedup 1.0000x reference)
import jax
import jax.numpy as jnp
from jax.experimental import pallas as pl


def kernel(x, pre_x, range_weight, pre_range_weight):
    raise NotImplementedError("write your pallas kernel here")


# speedup vs baseline: 3.7500x; 3.7500x over previous
import jax
import jax.numpy as jnp
from jax.experimental import pallas as pl

_SEARCH = 5
_PRE = 7
_KNN = 9
_CIN = 5
_STEM = 32
_BH = 8
_BW = 128


def _select_gather(diffs, val, k):
    n = len(diffs)
    d = list(diffs)
    ais = []
    for _ in range(k):
        mv = d[0]
        ai = jnp.zeros_like(mv)
        for s in range(1, n):
            lt = d[s] < mv
            mv = jnp.where(lt, d[s], mv)
            ai = jnp.where(lt, jnp.float32(s), ai)
        ais.append(ai)
        d = [jnp.where(ai == jnp.float32(s), jnp.float32(jnp.inf), ds)
             for s, ds in enumerate(d)]
    g = [[None] * k for _ in range(_CIN)]
    for s in range(n):
        vs = [val(c, s) for c in range(_CIN)]
        for j in range(k):
            m = ais[j] == jnp.float32(s)
            for c in range(_CIN):
                prev = g[c][j]
                g[c][j] = jnp.where(m, vs[c], 0.0 if prev is None else prev)
    return g


def _body(xp_ref, pxp_ref, w_ref, pw_ref, out_ref, pout_ref):
    b = pl.program_id(0)
    hb = pl.program_id(1)
    wb = pl.program_id(2)
    h0 = hb * _BH
    w0 = wb * _BW

    halx = [xp_ref[b, c, pl.ds(h0, 2 * _BH), pl.ds(w0, _BW + 128)]
            for c in range(_CIN)]
    halp = [pxp_ref[b, c, pl.ds(h0, 2 * _BH), pl.ds(w0, _BW + 128)]
            for c in range(_CIN)]

    def xs(c, di, dj):
        return jax.lax.slice(halx[c], (di, dj), (di + _BH, dj + _BW))

    def ps(c, di, dj):
        return jax.lax.slice(halp[c], (di, dj), (di + _BH, dj + _BW))

    center = xs(0, 3, 3)

    sd = []
    for s in range(_SEARCH * _SEARCH):
        di, dj = divmod(s, _SEARCH)
        if s == (_SEARCH * _SEARCH - 1) // 2:
            sd.append(jnp.full((_BH, _BW), -1.0, jnp.float32))
        else:
            sd.append(jnp.abs(xs(0, di + 1, dj + 1) - center))
    g_in = _select_gather(sd, lambda c, s: xs(c, 1 + s // _SEARCH, 1 + s % _SEARCH), _KNN)

    pd = []
    for s in range(_PRE * _PRE):
        di, dj = divmod(s, _PRE)
        pd.append(jnp.abs(ps(0, di, dj) - center))
    g_pre = _select_gather(pd, lambda c, s: ps(c, s // _PRE, s % _PRE), _KNN)

    ax = xs(1, 3, 3)
    ay = xs(2, 3, 3)
    az = xs(3, 3, 3)
    d_r, d_t, d_p = [], [], []
    for j in range(_KNN):
        x0 = g_pre[1][j] - ax
        y0 = g_pre[2][j] - ay
        z0 = g_pre[3][j] - az
        xy = x0 * x0 + y0 * y0
        z2 = z0 * z0
        r = jnp.sqrt(xy + z2)
        t = jnp.arctan2(jnp.sqrt(xy), z2)
        p = jnp.arctan2(t * t, r * r)
        d_r.append(r)
        d_t.append(t)
        d_p.append(p)

    rows_in = [g_in[c][j] for c in range(_CIN) for j in range(_KNN)]
    rows_pre = g_pre[0] + d_r + d_t + d_p + g_pre[4]

    gi = jnp.stack(rows_in).reshape(_CIN * _KNN, _BH * _BW)
    gp = jnp.stack(rows_pre).reshape(_CIN * _KNN, _BH * _BW)
    o1 = jnp.maximum(jnp.dot(w_ref[...], gi, preferred_element_type=jnp.float32), 0.0)
    o2 = jnp.maximum(jnp.dot(pw_ref[...], gp, preferred_element_type=jnp.float32), 0.0)
    out_ref[0] = o1.reshape(_STEM, _BH, _BW)
    pout_ref[0] = o2.reshape(_STEM, _BH, _BW)


def kernel(x, pre_x, range_weight, pre_range_weight):
    B, C, H, W = x.shape
    pad = (_PRE - 1) // 2
    hpad2 = (2 * _BH - _BH) + 8 - pad
    wpad2 = 256 - pad
    xp = jnp.pad(x, ((0, 0), (0, 0), (pad, hpad2), (pad, wpad2)))
    pxp = jnp.pad(pre_x, ((0, 0), (0, 0), (pad, hpad2), (pad, wpad2)))
    w1 = range_weight.reshape(_STEM, _CIN * _KNN)
    w2 = pre_range_weight.reshape(_STEM, _CIN * _KNN)

    grid = (B, H // _BH, W // _BW)
    out_sds = jax.ShapeDtypeStruct((B, _STEM, H, W), jnp.float32)
    in_specs = [
        pl.BlockSpec(xp.shape, lambda b, h, w: (0, 0, 0, 0)),
        pl.BlockSpec(pxp.shape, lambda b, h, w: (0, 0, 0, 0)),
        pl.BlockSpec(w1.shape, lambda b, h, w: (0, 0)),
        pl.BlockSpec(w2.shape, lambda b, h, w: (0, 0)),
    ]
    out_specs = [
        pl.BlockSpec((1, _STEM, _BH, _BW), lambda b, h, w: (b, 0, h, w)),
        pl.BlockSpec((1, _STEM, _BH, _BW), lambda b, h, w: (b, 0, h, w)),
    ]
    out, pre_out = pl.pallas_call(
        _body,
        grid=grid,
        in_specs=in_specs,
        out_specs=out_specs,
        out_shape=[out_sds, out_sds],
    )(xp, pxp, w1, w2)
    return (out, pre_out)


# speedup vs baseline: 8.9140x; 2.3771x over previous
import jax
import jax.numpy as jnp
from jax.experimental import pallas as pl
from jax.experimental.pallas import tpu as pltpu

_SEARCH = 5
_PRE = 7
_KNN = 9
_CIN = 5
_STEM = 32
_BH = 8
_BW = 128
_HALO_H = 2 * _BH
_HALO_W = _BW + 128
_CENTER = (_SEARCH * _SEARCH - 1) // 2
_GROUPS = ((0, 1), (2, 3), (4,))
_NROWS = 48


def _tree_argmin(slots, d):
    nodes = [(d[s], None, s) for s in slots]
    while len(nodes) > 1:
        nxt = []
        for a in range(0, len(nodes) - 1, 2):
            vl, il, cl = nodes[a]
            vr, ir, cr = nodes[a + 1]
            lt = vr < vl
            v = jnp.where(lt, vr, vl)
            ilv = jnp.full_like(vl, jnp.float32(cl)) if il is None else il
            irv = jnp.full_like(vr, jnp.float32(cr)) if ir is None else ir
            nxt.append((v, jnp.where(lt, irv, ilv), None))
        if len(nodes) % 2:
            nxt.append(nodes[-1])
        nodes = nxt
    v, i, c = nodes[0]
    return jnp.full_like(v, jnp.float32(c)) if i is None else i


def _topk_indices(slots, d, k):
    ais = []
    dd = dict(d)
    for j in range(k):
        ai = _tree_argmin(slots, dd)
        ais.append(ai)
        if j + 1 < k:
            dd = {s: jnp.where(ai == jnp.float32(s), jnp.float32(jnp.inf), dd[s])
                  for s in slots}
    return ais


def _body(xp_ref, pxp_ref, w_ref, pw_ref, out_ref, pout_ref, gi_scr, gp_scr):
    b = pl.program_id(0)
    hb = pl.program_id(1)
    wb = pl.program_id(2)
    h0 = hb * _BH
    w0 = wb * _BW

    def halo(ref, c):
        return ref[b, c, pl.ds(h0, _HALO_H), pl.ds(w0, _HALO_W)]

    def rrows(hal, di):
        return jax.lax.slice(hal, (di, 0), (di + _BH, _HALO_W))

    def lanes(row, dj):
        return jax.lax.slice(row, (0, dj), (_BH, dj + _BW))

    zeros = jnp.zeros((_BH, _BW), jnp.float32)

    def run_search(src_ref, knn, center):
        hal0 = halo(src_ref, 0)
        if src_ref is xp_ref:
            rows = {di: rrows(hal0, di + 1) for di in range(_SEARCH)}
            d = {}
            for s in range(_SEARCH * _SEARCH):
                if s == _CENTER:
                    continue
                di, dj = divmod(s, _SEARCH)
                d[s] = jnp.abs(lanes(rows[di], dj + 1) - center)
            return _topk_indices(sorted(d), d, knn)
        rows = {di: rrows(hal0, di) for di in range(_PRE)}
        d = {}
        for s in range(_PRE * _PRE):
            di, dj = divmod(s, _PRE)
            d[s] = jnp.abs(lanes(rows[di], dj) - center)
        return _topk_indices(sorted(d), d, knn)

    def run_gather(src_ref, ais, scr, row_of, search):
        k = _SEARCH if search else _PRE
        roff = 1 if search else 0
        for group in _GROUPS:
            rows = {}
            for c in group:
                hal = halo(src_ref, c)
                for di in range(k):
                    rows[c, di] = rrows(hal, di + roff)
            if search:
                for c in group:
                    scr[row_of(c, 0)] = lanes(rows[c, 2], 3)
            acc = {}
            for s in range(k * k):
                if search and s == _CENTER:
                    continue
                di, dj = divmod(s, k)
                vals = {c: lanes(rows[c, di], dj + roff) for c in group}
                for j, ai in enumerate(ais):
                    m = ai == jnp.float32(s)
                    for c in group:
                        prev = acc.get((c, j))
                        acc[c, j] = jnp.where(m, vals[c], zeros if prev is None else prev)
            jbase = 1 if search else 0
            for (c, j), v in acc.items():
                scr[row_of(c, j + jbase)] = v

    def matmul_out(scr, wref, oref):
        scr[45] = zeros
        scr[46] = zeros
        scr[47] = zeros
        g = scr[...].reshape(_NROWS, _BH * _BW)
        o = jnp.maximum(jnp.dot(wref[...], g, preferred_element_type=jnp.float32), 0.0)
        oref[0] = o.reshape(_STEM, _BH, _BW)

    center = lanes(rrows(halo(xp_ref, 0), 3), 3)
    s_ais = run_search(xp_ref, _KNN - 1, center)
    run_gather(xp_ref, s_ais, gi_scr, lambda c, j: c * _KNN + j, True)
    matmul_out(gi_scr, w_ref, out_ref)

    p_ais = run_search(pxp_ref, _KNN, center)
    run_gather(pxp_ref, p_ais, gp_scr, lambda c, j: c * _KNN + j, False)

    ax = gi_scr[1 * _KNN]
    ay = gi_scr[2 * _KNN]
    az = gi_scr[3 * _KNN]
    for j in range(_KNN):
        x0 = gp_scr[1 * _KNN + j] - ax
        y0 = gp_scr[2 * _KNN + j] - ay
        z0 = gp_scr[3 * _KNN + j] - az
        xy = x0 * x0 + y0 * y0
        z2 = z0 * z0
        r = jnp.sqrt(xy + z2)
        t = jnp.arctan2(jnp.sqrt(xy), z2)
        gp_scr[1 * _KNN + j] = r
        gp_scr[2 * _KNN + j] = t
        gp_scr[3 * _KNN + j] = jnp.arctan2(t * t, r * r)
    matmul_out(gp_scr, pw_ref, pout_ref)


def kernel(x, pre_x, range_weight, pre_range_weight):
    B, C, H, W = x.shape
    pad = (_PRE - 1) // 2
    hpad2 = _HALO_H - pad
    wpad2 = _HALO_W + 128 - _BW - pad
    xp = jnp.pad(x, ((0, 0), (0, 0), (pad, hpad2), (pad, wpad2)))
    pxp = jnp.pad(pre_x, ((0, 0), (0, 0), (pad, hpad2), (pad, wpad2)))
    w1 = jnp.pad(range_weight.reshape(_STEM, _CIN * _KNN), ((0, 0), (0, _NROWS - _CIN * _KNN)))
    w2 = jnp.pad(pre_range_weight.reshape(_STEM, _CIN * _KNN), ((0, 0), (0, _NROWS - _CIN * _KNN)))

    grid = (B, H // _BH, W // _BW)
    out_sds = jax.ShapeDtypeStruct((B, _STEM, H, W), jnp.float32)
    in_specs = [
        pl.BlockSpec(xp.shape, lambda b, h, w: (0, 0, 0, 0)),
        pl.BlockSpec(pxp.shape, lambda b, h, w: (0, 0, 0, 0)),
        pl.BlockSpec(w1.shape, lambda b, h, w: (0, 0)),
        pl.BlockSpec(w2.shape, lambda b, h, w: (0, 0)),
    ]
    out_specs = [
        pl.BlockSpec((1, _STEM, _BH, _BW), lambda b, h, w: (b, 0, h, w)),
        pl.BlockSpec((1, _STEM, _BH, _BW), lambda b, h, w: (b, 0, h, w)),
    ]
    out, pre_out = pl.pallas_call(
        _body,
        grid=grid,
        in_specs=in_specs,
        out_specs=out_specs,
        out_shape=[out_sds, out_sds],
        scratch_shapes=[
            pltpu.VMEM((_NROWS, _BH, _BW), jnp.float32),
            pltpu.VMEM((_NROWS, _BH, _BW), jnp.float32),
        ],
    )(xp, pxp, w1, w2)
    return (out, pre_out)
